# Initial kernel scaffold; baseline (speedup 1.0000x reference)
import jax
import jax.numpy as jnp
from jax.experimental import pallas as pl


def kernel(x, weight, bias):
    raise NotImplementedError("write your pallas kernel here")


# speedup vs baseline: 2.8511x; 2.8511x over previous
import functools

import jax
import jax.numpy as jnp
from jax import lax
from jax.experimental import pallas as pl
from jax.experimental.pallas import tpu as pltpu

_T = 16384
_DIM = 4096
_E = 64
_K = 8
_G = 8
_GS = _E // _G
_TG = 4
_SCALE = 2.5
_BT = 512


def _gate_body(x_ref, w_ref, b_ref, wout_ref, iout_ref):
    x = x_ref[...]
    w = w_ref[...]
    s = lax.dot_general(x, w, (((1,), (1,)), ((), ())),
                        preferred_element_type=jnp.float32)
    m = jnp.max(s, axis=-1, keepdims=True)
    e = jnp.exp(s - m)
    probs = e / jnp.sum(e, axis=-1, keepdims=True)
    biased = probs + b_ref[...]

    lane = lax.broadcasted_iota(jnp.int32, (_BT, _E), 1)
    neg_inf = jnp.float32(-jnp.inf)

    gscore_cols = []
    ingroup_masks = []
    for g in range(_G):
        ingroup = (lane // _GS) == g
        ingroup_masks.append(ingroup)
        mg = jnp.where(ingroup, biased, neg_inf)
        a1 = jnp.argmax(mg, axis=-1).astype(jnp.int32)[:, None]
        m1 = jnp.max(mg, axis=-1, keepdims=True)
        mg2 = jnp.where(lane == a1, neg_inf, mg)
        m2 = jnp.max(mg2, axis=-1, keepdims=True)
        gscore_cols.append(m1 + m2)
    gscore = jnp.concatenate(gscore_cols, axis=1)

    gidx = lax.broadcasted_iota(jnp.int32, (_BT, _G), 1)
    sel = jnp.zeros((_BT, _G), dtype=jnp.bool_)
    gs = gscore
    for _ in range(_TG):
        a = jnp.argmax(gs, axis=-1).astype(jnp.int32)[:, None]
        hit = gidx == a
        sel = sel | hit
        gs = jnp.where(hit, neg_inf, gs)

    allowed = jnp.zeros((_BT, _E), dtype=jnp.bool_)
    for g in range(_G):
        allowed = allowed | (ingroup_masks[g] & sel[:, g:g + 1])
    ms = jnp.where(allowed, biased, neg_inf)

    idx_cols = []
    w_cols = []
    for _ in range(_K):
        a = jnp.argmax(ms, axis=-1).astype(jnp.int32)[:, None]
        hit = lane == a
        wv = jnp.sum(jnp.where(hit, probs, 0.0), axis=-1, keepdims=True)
        idx_cols.append(a)
        w_cols.append(wv)
        ms = jnp.where(hit, neg_inf, ms)

    wout_ref[...] = jnp.concatenate(w_cols, axis=1) * jnp.float32(_SCALE)
    iout_ref[...] = jnp.concatenate(idx_cols, axis=1)


@jax.jit
def kernel(x, weight, bias):
    bias2 = bias.reshape(1, _E)
    grid = (_T // _BT,)
    weights, indices = pl.pallas_call(
        _gate_body,
        grid=grid,
        in_specs=[
            pl.BlockSpec((_BT, _DIM), lambda i: (i, 0)),
            pl.BlockSpec((_E, _DIM), lambda i: (0, 0)),
            pl.BlockSpec((1, _E), lambda i: (0, 0)),
        ],
        out_specs=[
            pl.BlockSpec((_BT, _K), lambda i: (i, 0)),
            pl.BlockSpec((_BT, _K), lambda i: (i, 0)),
        ],
        out_shape=[
            jax.ShapeDtypeStruct((_T, _K), jnp.float32),
            jax.ShapeDtypeStruct((_T, _K), jnp.int32),
        ],
        compiler_params=pltpu.CompilerParams(
            dimension_semantics=("arbitrary",),
        ),
    )(x, weight, bias2)
    return weights.astype(x.dtype), indices


# speedup vs baseline: 5.8998x; 2.0693x over previous
import functools

import jax
import jax.numpy as jnp
from jax import lax
from jax.experimental import pallas as pl
from jax.experimental.pallas import tpu as pltpu

_T = 16384
_DIM = 4096
_E = 64
_K = 8
_G = 8
_GS = _E // _G
_TG = 4
_SCALE = 2.5
_BT = 512


def _gate_body(x_ref, w_ref, b_ref, wout_ref, iout_ref):
    x = x_ref[...]
    w = w_ref[...]
    s = lax.dot_general(w, x, (((1,), (1,)), ((), ())),
                        preferred_element_type=jnp.float32)
    neg_inf = jnp.float32(-jnp.inf)

    m = jnp.max(s, axis=0, keepdims=True)
    e = jnp.exp(s - m)
    probs = e / jnp.sum(e, axis=0, keepdims=True)
    biased = probs + b_ref[...]

    gscore_rows = []
    for g in range(_G):
        grp = biased[g * _GS:(g + 1) * _GS, :]
        m1 = jnp.max(grp, axis=0, keepdims=True)
        eq = grp == m1
        cnt = jnp.sum(eq.astype(jnp.float32), axis=0, keepdims=True)
        m2 = jnp.max(jnp.where(eq, neg_inf, grp), axis=0, keepdims=True)
        m2 = jnp.where(cnt > 1.5, m1, m2)
        gscore_rows.append(m1 + m2)
    gscore = jnp.concatenate(gscore_rows, axis=0)

    riota_g = lax.broadcasted_iota(jnp.int32, (_G, _BT), 0)
    sel = jnp.zeros((_G, _BT), dtype=jnp.bool_)
    gs = gscore
    for _ in range(_TG):
        mx = jnp.max(gs, axis=0, keepdims=True)
        a = jnp.min(jnp.where(gs == mx, riota_g, _E), axis=0, keepdims=True)
        hit = riota_g == a
        sel = sel | hit
        gs = jnp.where(hit, neg_inf, gs)

    ms_rows = []
    for g in range(_G):
        grp = biased[g * _GS:(g + 1) * _GS, :]
        ms_rows.append(jnp.where(sel[g:g + 1, :], grp, neg_inf))
    ms = jnp.concatenate(ms_rows, axis=0)

    riota_e = lax.broadcasted_iota(jnp.int32, (_E, _BT), 0)
    idx_rows = []
    w_rows = []
    for _ in range(_K):
        mx = jnp.max(ms, axis=0, keepdims=True)
        a = jnp.min(jnp.where(ms == mx, riota_e, _E), axis=0, keepdims=True)
        hit = riota_e == a
        wv = jnp.sum(jnp.where(hit, probs, 0.0), axis=0, keepdims=True)
        idx_rows.append(a)
        w_rows.append(wv)
        ms = jnp.where(hit, neg_inf, ms)

    wout_ref[...] = jnp.concatenate(w_rows, axis=0) * jnp.float32(_SCALE)
    iout_ref[...] = jnp.concatenate(idx_rows, axis=0)


@jax.jit
def kernel(x, weight, bias):
    bias2 = bias.reshape(_E, 1)
    grid = (_T // _BT,)
    wt, it = pl.pallas_call(
        _gate_body,
        grid=grid,
        in_specs=[
            pl.BlockSpec((_BT, _DIM), lambda i: (i, 0)),
            pl.BlockSpec((_E, _DIM), lambda i: (0, 0)),
            pl.BlockSpec((_E, 1), lambda i: (0, 0)),
        ],
        out_specs=[
            pl.BlockSpec((_K, _BT), lambda i: (0, i)),
            pl.BlockSpec((_K, _BT), lambda i: (0, i)),
        ],
        out_shape=[
            jax.ShapeDtypeStruct((_K, _T), jnp.float32),
            jax.ShapeDtypeStruct((_K, _T), jnp.int32),
        ],
        compiler_params=pltpu.CompilerParams(
            dimension_semantics=("arbitrary",),
        ),
    )(x, weight, bias2)
    return wt.T.astype(x.dtype), it.T


# speedup vs baseline: 6.7243x; 1.1397x over previous
import functools

import jax
import jax.numpy as jnp
from jax import lax
from jax.experimental import pallas as pl
from jax.experimental.pallas import tpu as pltpu

_T = 16384
_DIM = 4096
_E = 64
_K = 8
_G = 8
_GS = _E // _G
_TG = 4
_SCALE = 2.5
_BT = 1024


def _gate_body(x_ref, w_ref, b_ref, wout_ref, iout_ref):
    x = x_ref[...]
    w = w_ref[...]
    s = lax.dot_general(w, x, (((1,), (1,)), ((), ())),
                        preferred_element_type=jnp.float32)
    neg_inf = jnp.float32(-jnp.inf)

    m = jnp.max(s, axis=0, keepdims=True)
    e = jnp.exp(s - m)
    probs = e / jnp.sum(e, axis=0, keepdims=True)
    biased = probs + b_ref[...]

    gscore_rows = []
    for g in range(_G):
        grp = biased[g * _GS:(g + 1) * _GS, :]
        m1 = jnp.max(grp, axis=0, keepdims=True)
        eq = grp == m1
        cnt = jnp.sum(eq.astype(jnp.float32), axis=0, keepdims=True)
        m2 = jnp.max(jnp.where(eq, neg_inf, grp), axis=0, keepdims=True)
        m2 = jnp.where(cnt > 1.5, m1, m2)
        gscore_rows.append(m1 + m2)
    gscore = jnp.concatenate(gscore_rows, axis=0)

    riota_g = lax.broadcasted_iota(jnp.int32, (_G, _BT), 0)
    sel = jnp.zeros((_G, _BT), dtype=jnp.bool_)
    gs = gscore
    for _ in range(_TG):
        mx = jnp.max(gs, axis=0, keepdims=True)
        a = jnp.min(jnp.where(gs == mx, riota_g, _E), axis=0, keepdims=True)
        hit = riota_g == a
        sel = sel | hit
        gs = jnp.where(hit, neg_inf, gs)

    ms_rows = []
    for g in range(_G):
        grp = biased[g * _GS:(g + 1) * _GS, :]
        ms_rows.append(jnp.where(sel[g:g + 1, :], grp, neg_inf))
    ms = jnp.concatenate(ms_rows, axis=0)

    riota_e = lax.broadcasted_iota(jnp.int32, (_E, _BT), 0)
    idx_rows = []
    w_rows = []
    for _ in range(_K):
        mx = jnp.max(ms, axis=0, keepdims=True)
        a = jnp.min(jnp.where(ms == mx, riota_e, _E), axis=0, keepdims=True)
        hit = riota_e == a
        wv = jnp.sum(jnp.where(hit, probs, 0.0), axis=0, keepdims=True)
        idx_rows.append(a)
        w_rows.append(wv)
        ms = jnp.where(hit, neg_inf, ms)

    wout_ref[...] = jnp.concatenate(w_rows, axis=0) * jnp.float32(_SCALE)
    iout_ref[...] = jnp.concatenate(idx_rows, axis=0)


@jax.jit
def kernel(x, weight, bias):
    bias2 = bias.reshape(_E, 1)
    grid = (_T // _BT,)
    wt, it = pl.pallas_call(
        _gate_body,
        grid=grid,
        in_specs=[
            pl.BlockSpec((_BT, _DIM), lambda i: (i, 0)),
            pl.BlockSpec((_E, _DIM), lambda i: (0, 0)),
            pl.BlockSpec((_E, 1), lambda i: (0, 0)),
        ],
        out_specs=[
            pl.BlockSpec((_K, _BT), lambda i: (0, i)),
            pl.BlockSpec((_K, _BT), lambda i: (0, i)),
        ],
        out_shape=[
            jax.ShapeDtypeStruct((_K, _T), jnp.float32),
            jax.ShapeDtypeStruct((_K, _T), jnp.int32),
        ],
        compiler_params=pltpu.CompilerParams(
            dimension_semantics=("arbitrary",),
        ),
    )(x, weight, bias2)
    return wt.T.astype(x.dtype), it.T


# speedup vs baseline: 6.7276x; 1.0005x over previous
import functools

import jax
import jax.numpy as jnp
from jax import lax
from jax.experimental import pallas as pl
from jax.experimental.pallas import tpu as pltpu

_T = 16384
_DIM = 4096
_E = 64
_K = 8
_G = 8
_GS = _E // _G
_TG = 4
_SCALE = 2.5
_BT = 1024


def _gate_body(x_ref, w_ref, b_ref, wout_ref, iout_ref):
    x = x_ref[...]
    w = w_ref[...]
    s = lax.dot_general(w, x, (((1,), (1,)), ((), ())),
                        preferred_element_type=jnp.float32)
    neg_inf = jnp.float32(-jnp.inf)

    m = jnp.max(s, axis=0, keepdims=True)
    e = jnp.exp(s - m)
    probs = e / jnp.sum(e, axis=0, keepdims=True)
    biased = probs + b_ref[...]

    gscore_rows = []
    for g in range(_G):
        grp = biased[g * _GS:(g + 1) * _GS, :]
        m1 = jnp.max(grp, axis=0, keepdims=True)
        eq = grp == m1
        cnt = jnp.sum(eq.astype(jnp.float32), axis=0, keepdims=True)
        m2 = jnp.max(jnp.where(eq, neg_inf, grp), axis=0, keepdims=True)
        m2 = jnp.where(cnt > 1.5, m1, m2)
        gscore_rows.append(m1 + m2)
    gscore = jnp.concatenate(gscore_rows, axis=0)

    riota_g = lax.broadcasted_iota(jnp.int32, (_G, _BT), 0)
    sel = jnp.zeros((_G, _BT), dtype=jnp.bool_)
    gs = gscore
    for _ in range(_TG):
        mx = jnp.max(gs, axis=0, keepdims=True)
        a = jnp.min(jnp.where(gs == mx, riota_g, _E), axis=0, keepdims=True)
        hit = riota_g == a
        sel = sel | hit
        gs = jnp.where(hit, neg_inf, gs)

    ms_rows = []
    for g in range(_G):
        grp = biased[g * _GS:(g + 1) * _GS, :]
        ms_rows.append(jnp.where(sel[g:g + 1, :], grp, neg_inf))
    ms = jnp.concatenate(ms_rows, axis=0)

    riota_e = lax.broadcasted_iota(jnp.int32, (_E, _BT), 0)
    idx_rows = []
    w_rows = []
    for _ in range(_K):
        mx = jnp.max(ms, axis=0, keepdims=True)
        a = jnp.min(jnp.where(ms == mx, riota_e, _E), axis=0, keepdims=True)
        hit = riota_e == a
        wv = jnp.sum(jnp.where(hit, probs, 0.0), axis=0, keepdims=True)
        idx_rows.append(a)
        w_rows.append(wv)
        ms = jnp.where(hit, neg_inf, ms)

    wout_ref[...] = jnp.concatenate(w_rows, axis=0) * jnp.float32(_SCALE)
    iout_ref[...] = jnp.concatenate(idx_rows, axis=0)


@jax.jit
def kernel(x, weight, bias):
    bias2 = bias.reshape(_E, 1)
    grid = (_T // _BT,)
    wt, it = pl.pallas_call(
        _gate_body,
        grid=grid,
        in_specs=[
            pl.BlockSpec((_BT, _DIM), lambda i: (i, 0)),
            pl.BlockSpec((_E, _DIM), lambda i: (0, 0)),
            pl.BlockSpec((_E, 1), lambda i: (0, 0)),
        ],
        out_specs=[
            pl.BlockSpec((_K, _BT), lambda i: (0, i)),
            pl.BlockSpec((_K, _BT), lambda i: (0, i)),
        ],
        out_shape=[
            jax.ShapeDtypeStruct((_K, _T), jnp.float32),
            jax.ShapeDtypeStruct((_K, _T), jnp.int32),
        ],
        compiler_params=pltpu.CompilerParams(
            dimension_semantics=("parallel",),
            vmem_limit_bytes=100 * 1024 * 1024,
        ),
    )(x, weight, bias2)
    return wt.T.astype(x.dtype), it.T
